# baseline (speedup 1.0000x reference)
import functools

import jax
import jax.numpy as jnp
from jax import lax
from jax.experimental import pallas as pl
from jax.experimental.pallas import tpu as pltpu
from jax.experimental.pallas import tpu_sc as plsc

N = 10000
E = 320000
D = 128
COLH = 64

NC = 2
NS = 16
CHUNK = 512
EP = 327680
NCHUNKS = EP // CHUNK
CPT = NCHUNKS // NS
QCH = 20
NBUF = 2
ZCH = 128
ZROWS = 632
AGG_ROWS = NS * ZROWS
WB = 624
WB_LAST = N - (NS - 1) * WB


def _sc_agg_body(hcat, srcoff, dst2, out, *scr):
    src_st = scr[0]
    dst_st = scr[1]
    bufs = scr[2:2 + NBUF]
    agg_sh = scr[2 + NBUF]
    semg = scr[3 + NBUF:3 + 2 * NBUF]
    sems = scr[3 + 2 * NBUF:3 + 3 * NBUF]

    c = lax.axis_index("c")
    s = lax.axis_index("s")

    def zbody(i, carry):
        for j in range(COLH // 16):
            bufs[0][i, pl.ds(j * 16, 16)] = jnp.zeros((16,), jnp.float32)
        return carry
    lax.fori_loop(0, ZCH, zbody, 0)

    zbase = s * ZROWS
    for k in range(ZROWS // ZCH):
        pltpu.sync_copy(bufs[0].at[pl.ds(0, ZCH)],
                        agg_sh.at[pl.ds(zbase + k * ZCH, ZCH)])
    rem = ZROWS % ZCH
    if rem:
        pltpu.sync_copy(bufs[0].at[pl.ds(0, rem)],
                        agg_sh.at[pl.ds(zbase + (ZROWS // ZCH) * ZCH, rem)])
    plsc.subcore_barrier()

    def wait_gather(j, g):
        pltpu.make_async_copy(hcat.at[src_st.at[g]], bufs[j], semg[j]).wait()

    def wait_scatter(j):
        pltpu.make_async_copy(bufs[j], agg_sh.at[dst_st.at[0]], sems[j]).wait()

    cbase = s * CPT
    for q in range(CPT // QCH):
        if q > 0:
            for j in range(NBUF):
                wait_scatter(j)
        qb = cbase + q * QCH
        pltpu.sync_copy(srcoff.at[pl.ds(c * NCHUNKS + qb, QCH)], src_st)
        pltpu.sync_copy(dst2.at[pl.ds(qb, QCH)], dst_st)

        def qloop(gg, carry):
            base = gg * NBUF
            for j in range(NBUF):
                @pl.when(gg > 0)
                def _(j=j):
                    wait_scatter(j)
                pltpu.async_copy(hcat.at[src_st.at[base + j]], bufs[j], semg[j])
            for j in range(NBUF):
                wait_gather(j, base + j)
                pltpu.async_copy(bufs[j], agg_sh.at[dst_st.at[base + j]],
                                 sems[j], add=True)
            return carry

        lax.fori_loop(0, QCH // NBUF, qloop, 0)

    for j in range(NBUF):
        wait_scatter(j)

    plsc.subcore_barrier()

    @pl.when(s < NS - 1)
    def _():
        pltpu.sync_copy(agg_sh.at[pl.ds(s * WB, WB)],
                        out.at[pl.ds(c * N + s * WB, WB)])

    @pl.when(s == NS - 1)
    def _():
        pltpu.sync_copy(agg_sh.at[pl.ds((NS - 1) * WB, WB_LAST)],
                        out.at[pl.ds(c * N + (NS - 1) * WB, WB_LAST)])


def _sc_agg(hcat, srcoff, dst2):
    mesh = plsc.VectorSubcoreMesh(core_axis_name="c", subcore_axis_name="s",
                                  num_cores=NC, num_subcores=NS)
    return pl.kernel(
        _sc_agg_body,
        out_type=jax.ShapeDtypeStruct((2 * N, COLH), jnp.float32),
        mesh=mesh,
        compiler_params=pltpu.CompilerParams(use_tc_tiling_on_sc=False),
        scratch_types=[
            pltpu.VMEM((QCH, CHUNK), jnp.int32),
            pltpu.VMEM((QCH, CHUNK), jnp.int32),
        ] + [pltpu.VMEM((CHUNK, COLH), jnp.float32) for _ in range(NBUF)]
        + [pltpu.VMEM_SHARED((AGG_ROWS, COLH), jnp.float32)]
        + [pltpu.SemaphoreType.DMA for _ in range(2 * NBUF)],
    )(hcat, srcoff, dst2)


def _tc_mlp_body(relu, hl_ref, hr_ref, al_ref, ar_ref, w_ref, b_ref, o_ref):
    t = jnp.concatenate(
        [hl_ref[...] + al_ref[...], hr_ref[...] + ar_ref[...]], axis=1)
    y = jnp.dot(t, w_ref[0], preferred_element_type=jnp.float32) + b_ref[0]
    if relu:
        y = jnp.maximum(y, 0.0)
    o_ref[...] = y


def _tc_mlp(hcat, agg, w, b, relu):
    blk = 1000
    nb = N // blk
    return pl.pallas_call(
        functools.partial(_tc_mlp_body, relu),
        grid=(2 * nb,),
        in_specs=[
            pl.BlockSpec((blk, COLH), lambda i: (i % nb, 0)),
            pl.BlockSpec((blk, COLH), lambda i: (nb + i % nb, 0)),
            pl.BlockSpec((blk, COLH), lambda i: (i % nb, 0)),
            pl.BlockSpec((blk, COLH), lambda i: (nb + i % nb, 0)),
            pl.BlockSpec((1, D, COLH), lambda i: (i // nb, 0, 0)),
            pl.BlockSpec((1, 1, COLH), lambda i: (i // nb, 0, 0)),
        ],
        out_specs=pl.BlockSpec((blk, COLH), lambda i: (i, 0)),
        out_shape=jax.ShapeDtypeStruct((2 * N, COLH), jnp.float32),
    )(hcat, hcat, agg, agg,
      jnp.stack([w[:, :COLH], w[:, COLH:]]),
      b.reshape(2, 1, COLH))


def kernel(x, edge_index, W1, b1, W2, b2, W3, b3):
    src = edge_index[0].astype(jnp.int32)
    dst = edge_index[1].astype(jnp.int32)
    pad = EP - E
    src2 = jnp.concatenate([src, (jnp.arange(pad, dtype=jnp.int32) * 13) % N]
                           ).reshape(NCHUNKS, CHUNK)
    dst2 = jnp.concatenate([dst, jnp.full((pad,), N, jnp.int32)]).reshape(NCHUNKS, CHUNK)
    srcoff = jnp.concatenate([src2, src2 + N])

    hcat = jnp.concatenate([x[:, :COLH], x[:, COLH:]], axis=0)
    agg = _sc_agg(hcat, srcoff, dst2)
    hcat = _tc_mlp(hcat, agg, W1, b1, relu=True)
    agg = _sc_agg(hcat, srcoff, dst2)
    hcat = _tc_mlp(hcat, agg, W2, b2, relu=True)
    agg = _sc_agg(hcat, srcoff, dst2)
    w3p = jnp.zeros((D, D), jnp.float32).at[:, :40].set(W3)
    b3p = jnp.zeros((D,), jnp.float32).at[:40].set(b3)
    hcat = _tc_mlp(hcat, agg, w3p, b3p, relu=False)
    return hcat[:N, :40]


# speedup vs baseline: 1.1342x; 1.1342x over previous
import functools

import jax
import jax.numpy as jnp
from jax import lax
from jax.experimental import pallas as pl
from jax.experimental.pallas import tpu as pltpu
from jax.experimental.pallas import tpu_sc as plsc

N = 10000
E = 320000
D = 128
COLH = 64

NC = 2
NS = 16
CHUNK = 256
EP = 327680
NCHUNKS = EP // CHUNK
CPT = NCHUNKS // NS
SQ = 16
NBUF = 4
ZCH = 128
ZROWS = 632
AGG_ROWS = NS * ZROWS
WB = 624
WB_LAST = N - (NS - 1) * WB


def _sc_agg_body(hcat, srcoff, dst2, out, *scr):
    src_st = scr[0]
    dst_st = scr[1]
    bufs = scr[2:2 + NBUF]
    agg_sh = scr[2 + NBUF]
    semg = scr[3 + NBUF:3 + 2 * NBUF]
    sems = scr[3 + 2 * NBUF:3 + 3 * NBUF]

    c = lax.axis_index("c")
    s = lax.axis_index("s")

    def zbody(i, carry):
        for j in range(COLH // 16):
            bufs[0][i, pl.ds(j * 16, 16)] = jnp.zeros((16,), jnp.float32)
        return carry
    lax.fori_loop(0, ZCH, zbody, 0)

    zbase = s * ZROWS
    for k in range(ZROWS // ZCH):
        pltpu.sync_copy(bufs[0].at[pl.ds(0, ZCH)],
                        agg_sh.at[pl.ds(zbase + k * ZCH, ZCH)])
    rem = ZROWS % ZCH
    if rem:
        pltpu.sync_copy(bufs[0].at[pl.ds(0, rem)],
                        agg_sh.at[pl.ds(zbase + (ZROWS // ZCH) * ZCH, rem)])
    plsc.subcore_barrier()

    def wait_gather(j, g):
        pltpu.make_async_copy(hcat.at[src_st.at[g]], bufs[j], semg[j]).wait()

    def wait_scatter(j):
        pltpu.make_async_copy(bufs[j], agg_sh.at[dst_st.at[0]], sems[j]).wait()

    cbase = s * CPT
    pltpu.sync_copy(dst2.at[pl.ds(cbase, CPT)], dst_st)
    for q in range(CPT // SQ):
        pltpu.sync_copy(srcoff.at[pl.ds(c * NCHUNKS + cbase + q * SQ, SQ)],
                        src_st)

        def qloop(gg, carry, first=(q == 0)):
            base = gg * NBUF
            for j in range(NBUF):
                if first:
                    @pl.when(gg > 0)
                    def _(j=j):
                        wait_scatter(j)
                else:
                    wait_scatter(j)
                pltpu.async_copy(hcat.at[src_st.at[base + j]], bufs[j], semg[j])
            for j in range(NBUF):
                wait_gather(j, base + j)
                pltpu.async_copy(
                    bufs[j], agg_sh.at[dst_st.at[q * SQ + base + j]],
                    sems[j], add=True)
            return carry

        lax.fori_loop(0, SQ // NBUF, qloop, 0)

    for j in range(NBUF):
        wait_scatter(j)

    plsc.subcore_barrier()

    @pl.when(s < NS - 1)
    def _():
        pltpu.sync_copy(agg_sh.at[pl.ds(s * WB, WB)],
                        out.at[pl.ds(c * N + s * WB, WB)])

    @pl.when(s == NS - 1)
    def _():
        pltpu.sync_copy(agg_sh.at[pl.ds((NS - 1) * WB, WB_LAST)],
                        out.at[pl.ds(c * N + (NS - 1) * WB, WB_LAST)])


def _sc_agg(hcat, srcoff, dst2):
    mesh = plsc.VectorSubcoreMesh(core_axis_name="c", subcore_axis_name="s",
                                  num_cores=NC, num_subcores=NS)
    return pl.kernel(
        _sc_agg_body,
        out_type=jax.ShapeDtypeStruct((2 * N, COLH), jnp.float32),
        mesh=mesh,
        compiler_params=pltpu.CompilerParams(use_tc_tiling_on_sc=False),
        scratch_types=[
            pltpu.VMEM((SQ, CHUNK), jnp.int32),
            pltpu.VMEM((CPT, CHUNK), jnp.int32),
        ] + [pltpu.VMEM((CHUNK, COLH), jnp.float32) for _ in range(NBUF)]
        + [pltpu.VMEM_SHARED((AGG_ROWS, COLH), jnp.float32)]
        + [pltpu.SemaphoreType.DMA for _ in range(2 * NBUF)],
    )(hcat, srcoff, dst2)


def _tc_mlp_body(relu, hl_ref, hr_ref, al_ref, ar_ref, w_ref, b_ref, o_ref):
    t = jnp.concatenate(
        [hl_ref[...] + al_ref[...], hr_ref[...] + ar_ref[...]], axis=1)
    y = jnp.dot(t, w_ref[0], preferred_element_type=jnp.float32) + b_ref[0]
    if relu:
        y = jnp.maximum(y, 0.0)
    o_ref[...] = y


def _tc_mlp(hcat, agg, w, b, relu):
    blk = 1000
    nb = N // blk
    return pl.pallas_call(
        functools.partial(_tc_mlp_body, relu),
        grid=(2 * nb,),
        in_specs=[
            pl.BlockSpec((blk, COLH), lambda i: (i % nb, 0)),
            pl.BlockSpec((blk, COLH), lambda i: (nb + i % nb, 0)),
            pl.BlockSpec((blk, COLH), lambda i: (i % nb, 0)),
            pl.BlockSpec((blk, COLH), lambda i: (nb + i % nb, 0)),
            pl.BlockSpec((1, D, COLH), lambda i: (i // nb, 0, 0)),
            pl.BlockSpec((1, 1, COLH), lambda i: (i // nb, 0, 0)),
        ],
        out_specs=pl.BlockSpec((blk, COLH), lambda i: (i, 0)),
        out_shape=jax.ShapeDtypeStruct((2 * N, COLH), jnp.float32),
    )(hcat, hcat, agg, agg,
      jnp.stack([w[:, :COLH], w[:, COLH:]]),
      b.reshape(2, 1, COLH))


def kernel(x, edge_index, W1, b1, W2, b2, W3, b3):
    src = edge_index[0].astype(jnp.int32)
    dst = edge_index[1].astype(jnp.int32)
    pad = EP - E
    src2 = jnp.concatenate([src, (jnp.arange(pad, dtype=jnp.int32) * 13) % N]
                           ).reshape(NCHUNKS, CHUNK)
    dst2 = jnp.concatenate([dst, jnp.full((pad,), N, jnp.int32)]).reshape(NCHUNKS, CHUNK)
    srcoff = jnp.concatenate([src2, src2 + N])

    hcat = jnp.concatenate([x[:, :COLH], x[:, COLH:]], axis=0)
    agg = _sc_agg(hcat, srcoff, dst2)
    hcat = _tc_mlp(hcat, agg, W1, b1, relu=True)
    agg = _sc_agg(hcat, srcoff, dst2)
    hcat = _tc_mlp(hcat, agg, W2, b2, relu=True)
    agg = _sc_agg(hcat, srcoff, dst2)
    w3p = jnp.zeros((D, D), jnp.float32).at[:, :40].set(W3)
    b3p = jnp.zeros((D,), jnp.float32).at[:40].set(b3)
    hcat = _tc_mlp(hcat, agg, w3p, b3p, relu=False)
    return hcat[:N, :40]


# speedup vs baseline: 1.1785x; 1.0391x over previous
import functools

import jax
import jax.numpy as jnp
from jax import lax
from jax.experimental import pallas as pl
from jax.experimental.pallas import tpu as pltpu
from jax.experimental.pallas import tpu_sc as plsc

N = 10000
E = 320000
D = 128
COLH = 64

NC = 2
NS = 16
CHUNK = 256
EP = 327680
NCHUNKS = EP // CHUNK
CPT = NCHUNKS // NS
SQ = 16
NBUF = 4
ZCH = 128
ZROWS = 632
AGG_ROWS = NS * ZROWS
WB = 624
WB_LAST = N - (NS - 1) * WB


def _sc_agg_body(hcat, srcoff, dst2, out, *scr):
    src_st = scr[0]
    dst_st = scr[1]
    bufs = scr[2:2 + NBUF]
    agg_sh = scr[2 + NBUF]
    semg = scr[3 + NBUF:3 + 2 * NBUF]
    sems = scr[3 + 2 * NBUF:3 + 3 * NBUF]

    c = lax.axis_index("c")
    s = lax.axis_index("s")

    def zbody(i, carry):
        for j in range(COLH // 16):
            bufs[0][i, pl.ds(j * 16, 16)] = jnp.zeros((16,), jnp.float32)
        return carry
    lax.fori_loop(0, ZCH, zbody, 0)

    zbase = s * ZROWS
    for k in range(ZROWS // ZCH):
        pltpu.sync_copy(bufs[0].at[pl.ds(0, ZCH)],
                        agg_sh.at[pl.ds(zbase + k * ZCH, ZCH)])
    rem = ZROWS % ZCH
    if rem:
        pltpu.sync_copy(bufs[0].at[pl.ds(0, rem)],
                        agg_sh.at[pl.ds(zbase + (ZROWS // ZCH) * ZCH, rem)])
    plsc.subcore_barrier()

    def wait_gather(j, g):
        pltpu.make_async_copy(hcat.at[src_st.at[g]], bufs[j], semg[j]).wait()

    def wait_scatter(j):
        pltpu.make_async_copy(bufs[j], agg_sh.at[dst_st.at[0]], sems[j]).wait()

    cbase = s * CPT
    pltpu.sync_copy(dst2.at[pl.ds(cbase, CPT)], dst_st)
    for q in range(CPT // SQ):
        pltpu.sync_copy(srcoff.at[pl.ds(c * NCHUNKS + cbase + q * SQ, SQ)],
                        src_st)

        def qloop(gg, carry, first=(q == 0)):
            base = gg * NBUF
            for j in range(NBUF):
                if first:
                    @pl.when(gg > 0)
                    def _(j=j):
                        wait_scatter(j)
                else:
                    wait_scatter(j)
                pltpu.async_copy(hcat.at[src_st.at[base + j]], bufs[j], semg[j])
            for j in range(NBUF):
                wait_gather(j, base + j)
                pltpu.async_copy(
                    bufs[j], agg_sh.at[dst_st.at[q * SQ + base + j]],
                    sems[j], add=True)
            return carry

        lax.fori_loop(0, SQ // NBUF, qloop, 0)

    for j in range(NBUF):
        wait_scatter(j)

    plsc.subcore_barrier()

    @pl.when(s < NS - 1)
    def _():
        pltpu.sync_copy(agg_sh.at[pl.ds(s * WB, WB)],
                        out.at[pl.ds(c * N + s * WB, WB)])

    @pl.when(s == NS - 1)
    def _():
        pltpu.sync_copy(agg_sh.at[pl.ds((NS - 1) * WB, WB_LAST)],
                        out.at[pl.ds(c * N + (NS - 1) * WB, WB_LAST)])


def _sc_agg(hcat, srcoff, dst2):
    mesh = plsc.VectorSubcoreMesh(core_axis_name="c", subcore_axis_name="s",
                                  num_cores=NC, num_subcores=NS)
    return pl.kernel(
        _sc_agg_body,
        out_type=jax.ShapeDtypeStruct((2 * N, COLH), jnp.float32),
        mesh=mesh,
        compiler_params=pltpu.CompilerParams(use_tc_tiling_on_sc=False),
        scratch_types=[
            pltpu.VMEM((SQ, CHUNK), jnp.int32),
            pltpu.VMEM((CPT, CHUNK), jnp.int32),
        ] + [pltpu.VMEM((CHUNK, COLH), jnp.float32) for _ in range(NBUF)]
        + [pltpu.VMEM_SHARED((AGG_ROWS, COLH), jnp.float32)]
        + [pltpu.SemaphoreType.DMA for _ in range(2 * NBUF)],
    )(hcat, srcoff, dst2)


def _tc_mlp_body(relu, hl_ref, hr_ref, al_ref, ar_ref, w_ref, b_ref, o_ref):
    t = jnp.concatenate(
        [hl_ref[...] + al_ref[...], hr_ref[...] + ar_ref[...]], axis=1)
    y = jnp.dot(t, w_ref[0], preferred_element_type=jnp.float32) + b_ref[0]
    if relu:
        y = jnp.maximum(y, 0.0)
    o_ref[...] = y


def _tc_mlp(hcat, agg, w, b, relu):
    blk = 2000
    nb = N // blk
    return pl.pallas_call(
        functools.partial(_tc_mlp_body, relu),
        grid=(2 * nb,),
        in_specs=[
            pl.BlockSpec((blk, COLH), lambda i: (i % nb, 0)),
            pl.BlockSpec((blk, COLH), lambda i: (nb + i % nb, 0)),
            pl.BlockSpec((blk, COLH), lambda i: (i % nb, 0)),
            pl.BlockSpec((blk, COLH), lambda i: (nb + i % nb, 0)),
            pl.BlockSpec((1, D, COLH), lambda i: (i // nb, 0, 0)),
            pl.BlockSpec((1, 1, COLH), lambda i: (i // nb, 0, 0)),
        ],
        out_specs=pl.BlockSpec((blk, COLH), lambda i: (i, 0)),
        out_shape=jax.ShapeDtypeStruct((2 * N, COLH), jnp.float32),
    )(hcat, hcat, agg, agg,
      jnp.stack([w[:, :COLH], w[:, COLH:]]),
      b.reshape(2, 1, COLH))


def kernel(x, edge_index, W1, b1, W2, b2, W3, b3):
    src = edge_index[0].astype(jnp.int32)
    dst = edge_index[1].astype(jnp.int32)
    pad = EP - E
    src2 = jnp.concatenate([src, (jnp.arange(pad, dtype=jnp.int32) * 13) % N]
                           ).reshape(NCHUNKS, CHUNK)
    dst2 = jnp.concatenate([dst, jnp.full((pad,), N, jnp.int32)]).reshape(NCHUNKS, CHUNK)
    srcoff = jnp.concatenate([src2, src2 + N])

    hcat = jnp.concatenate([x[:, :COLH], x[:, COLH:]], axis=0)
    agg = _sc_agg(hcat, srcoff, dst2)
    hcat = _tc_mlp(hcat, agg, W1, b1, relu=True)
    agg = _sc_agg(hcat, srcoff, dst2)
    hcat = _tc_mlp(hcat, agg, W2, b2, relu=True)
    agg = _sc_agg(hcat, srcoff, dst2)
    w3p = jnp.zeros((D, D), jnp.float32).at[:, :40].set(W3)
    b3p = jnp.zeros((D,), jnp.float32).at[:40].set(b3)
    hcat = _tc_mlp(hcat, agg, w3p, b3p, relu=False)
    return hcat[:N, :40]


# speedup vs baseline: 1.3247x; 1.1241x over previous
import functools

import jax
import jax.numpy as jnp
from jax import lax
from jax.experimental import pallas as pl
from jax.experimental.pallas import tpu as pltpu
from jax.experimental.pallas import tpu_sc as plsc

N = 10000
E = 320000
D = 128
COLH = 64

NC = 2
NS = 16
CHUNK = 256
EP = 327680
NCHUNKS = EP // CHUNK
CPT = NCHUNKS // NS
SQ = 16
NBUF = 4
ZCH = 128
ZROWS = 632
AGG_ROWS = NS * ZROWS
WB = 624
WB_LAST = N - (NS - 1) * WB


def _sc_agg_body(mode3, hcat, srcoff, dst2, out, *scr):
    src_st = scr[0]
    dst_st = scr[1]
    bufs = scr[2:2 + NBUF]
    agg_sh = scr[2 + NBUF]
    semg = scr[3 + NBUF:3 + 2 * NBUF]
    sems = scr[3 + 2 * NBUF:3 + 3 * NBUF]

    c = lax.axis_index("c")
    s = lax.axis_index("s")

    if mode3:
        cpt = NCHUNKS // 2 // NS
        sq = SQ // 2
        dst_base = c * (NCHUNKS // 2) + s * cpt
        src_base = dst_base
    else:
        cpt = CPT
        sq = SQ
        dst_base = s * cpt
        src_base = c * NCHUNKS + s * cpt

    def zbody(i, carry):
        for j in range(COLH // 16):
            bufs[0][i, pl.ds(j * 16, 16)] = jnp.zeros((16,), jnp.float32)
        return carry
    lax.fori_loop(0, ZCH, zbody, 0)

    zbase = s * ZROWS
    for k in range(ZROWS // ZCH):
        pltpu.sync_copy(bufs[0].at[pl.ds(0, ZCH)],
                        agg_sh.at[pl.ds(zbase + k * ZCH, ZCH)])
    rem = ZROWS % ZCH
    if rem:
        pltpu.sync_copy(bufs[0].at[pl.ds(0, rem)],
                        agg_sh.at[pl.ds(zbase + (ZROWS // ZCH) * ZCH, rem)])
    plsc.subcore_barrier()

    def wait_gather(j, g):
        pltpu.make_async_copy(hcat.at[src_st.at[g]], bufs[j], semg[j]).wait()

    def wait_scatter(j):
        pltpu.make_async_copy(bufs[j], agg_sh.at[dst_st.at[0]], sems[j]).wait()

    pltpu.sync_copy(dst2.at[pl.ds(dst_base, cpt)], dst_st.at[pl.ds(0, cpt)])
    for q in range(cpt // sq):
        pltpu.sync_copy(srcoff.at[pl.ds(src_base + q * sq, sq)],
                        src_st.at[pl.ds(0, sq)])

        def qloop(gg, carry, first=(q == 0)):
            base = gg * NBUF
            for j in range(NBUF):
                if first:
                    @pl.when(gg > 0)
                    def _(j=j):
                        wait_scatter(j)
                else:
                    wait_scatter(j)
                pltpu.async_copy(hcat.at[src_st.at[base + j]], bufs[j], semg[j])
            for j in range(NBUF):
                wait_gather(j, base + j)
                pltpu.async_copy(
                    bufs[j], agg_sh.at[dst_st.at[q * sq + base + j]],
                    sems[j], add=True)
            return carry

        lax.fori_loop(0, sq // NBUF, qloop, 0)

    for j in range(NBUF):
        wait_scatter(j)

    plsc.subcore_barrier()

    @pl.when(s < NS - 1)
    def _():
        pltpu.sync_copy(agg_sh.at[pl.ds(s * WB, WB)],
                        out.at[pl.ds(c * N + s * WB, WB)])

    @pl.when(s == NS - 1)
    def _():
        pltpu.sync_copy(agg_sh.at[pl.ds((NS - 1) * WB, WB_LAST)],
                        out.at[pl.ds(c * N + (NS - 1) * WB, WB_LAST)])


def _sc_agg(hcat, srcoff, dst2, mode3=False):
    mesh = plsc.VectorSubcoreMesh(core_axis_name="c", subcore_axis_name="s",
                                  num_cores=NC, num_subcores=NS)
    return pl.kernel(
        functools.partial(_sc_agg_body, mode3),
        out_type=jax.ShapeDtypeStruct((2 * N, COLH), jnp.float32),
        mesh=mesh,
        compiler_params=pltpu.CompilerParams(use_tc_tiling_on_sc=False),
        scratch_types=[
            pltpu.VMEM((SQ, CHUNK), jnp.int32),
            pltpu.VMEM((CPT, CHUNK), jnp.int32),
        ] + [pltpu.VMEM((CHUNK, COLH), jnp.float32) for _ in range(NBUF)]
        + [pltpu.VMEM_SHARED((AGG_ROWS, COLH), jnp.float32)]
        + [pltpu.SemaphoreType.DMA for _ in range(2 * NBUF)],
    )(hcat, srcoff, dst2)


def _tc_mlp_body(relu, hl_ref, hr_ref, al_ref, ar_ref, w_ref, b_ref, o_ref):
    t = jnp.concatenate(
        [hl_ref[...] + al_ref[...], hr_ref[...] + ar_ref[...]], axis=1)
    y = jnp.dot(t, w_ref[0], preferred_element_type=jnp.float32) + b_ref[0]
    if relu:
        y = jnp.maximum(y, 0.0)
    o_ref[...] = y


def _tc_mlp(hcat, agg, w, b, relu):
    blk = 2000
    nb = N // blk
    return pl.pallas_call(
        functools.partial(_tc_mlp_body, relu),
        grid=(2 * nb,),
        in_specs=[
            pl.BlockSpec((blk, COLH), lambda i: (i % nb, 0)),
            pl.BlockSpec((blk, COLH), lambda i: (nb + i % nb, 0)),
            pl.BlockSpec((blk, COLH), lambda i: (i % nb, 0)),
            pl.BlockSpec((blk, COLH), lambda i: (nb + i % nb, 0)),
            pl.BlockSpec((1, D, COLH), lambda i: (i // nb, 0, 0)),
            pl.BlockSpec((1, 1, COLH), lambda i: (i // nb, 0, 0)),
        ],
        out_specs=pl.BlockSpec((blk, COLH), lambda i: (i, 0)),
        out_shape=jax.ShapeDtypeStruct((2 * N, COLH), jnp.float32),
    )(hcat, hcat, agg, agg,
      jnp.stack([w[:, :COLH], w[:, COLH:]]),
      b.reshape(2, 1, COLH))


def _tc_transform_body(hl_ref, hr_ref, w_ref, o_ref):
    t = jnp.concatenate([hl_ref[...], hr_ref[...]], axis=1)
    o_ref[...] = jnp.dot(t, w_ref[...], preferred_element_type=jnp.float32)


def _tc_transform(hcat, w):
    blk = 2000
    nb = N // blk
    return pl.pallas_call(
        _tc_transform_body,
        grid=(nb,),
        in_specs=[
            pl.BlockSpec((blk, COLH), lambda i: (i, 0)),
            pl.BlockSpec((blk, COLH), lambda i: (nb + i, 0)),
            pl.BlockSpec((D, COLH), lambda i: (0, 0)),
        ],
        out_specs=pl.BlockSpec((blk, COLH), lambda i: (i, 0)),
        out_shape=jax.ShapeDtypeStruct((N, COLH), jnp.float32),
    )(hcat, hcat, w)


def _tc_combine_body(y_ref, a0_ref, a1_ref, b_ref, o_ref):
    o_ref[...] = y_ref[...] + a0_ref[...] + a1_ref[...] + b_ref[...]


def _tc_combine(y, agg, b):
    blk = 2000
    nb = N // blk
    return pl.pallas_call(
        _tc_combine_body,
        grid=(nb,),
        in_specs=[
            pl.BlockSpec((blk, COLH), lambda i: (i, 0)),
            pl.BlockSpec((blk, COLH), lambda i: (i, 0)),
            pl.BlockSpec((blk, COLH), lambda i: (nb + i, 0)),
            pl.BlockSpec((1, COLH), lambda i: (0, 0)),
        ],
        out_specs=pl.BlockSpec((blk, COLH), lambda i: (i, 0)),
        out_shape=jax.ShapeDtypeStruct((N, COLH), jnp.float32),
    )(y, agg, agg, b.reshape(1, COLH))


def kernel(x, edge_index, W1, b1, W2, b2, W3, b3):
    src = edge_index[0].astype(jnp.int32)
    dst = edge_index[1].astype(jnp.int32)
    pad = EP - E
    src2 = jnp.concatenate([src, (jnp.arange(pad, dtype=jnp.int32) * 13) % N]
                           ).reshape(NCHUNKS, CHUNK)
    dst2 = jnp.concatenate([dst, jnp.full((pad,), N, jnp.int32)]).reshape(NCHUNKS, CHUNK)
    srcoff = jnp.concatenate([src2, src2 + N])

    hcat = jnp.concatenate([x[:, :COLH], x[:, COLH:]], axis=0)
    agg = _sc_agg(hcat, srcoff, dst2)
    hcat = _tc_mlp(hcat, agg, W1, b1, relu=True)
    agg = _sc_agg(hcat, srcoff, dst2)
    hcat = _tc_mlp(hcat, agg, W2, b2, relu=True)
    w3p = jnp.zeros((D, COLH), jnp.float32).at[:, :40].set(W3)
    b3p = jnp.zeros((COLH,), jnp.float32).at[:40].set(b3)
    y = _tc_transform(hcat, w3p)
    agg = _sc_agg(y, srcoff, dst2, mode3=True)
    out = _tc_combine(y, agg, b3p)
    return out[:, :40]


# speedup vs baseline: 1.3256x; 1.0007x over previous
import functools

import jax
import jax.numpy as jnp
from jax import lax
from jax.experimental import pallas as pl
from jax.experimental.pallas import tpu as pltpu
from jax.experimental.pallas import tpu_sc as plsc

N = 10000
E = 320000
D = 128
COLH = 64

NC = 2
NS = 16
CHUNK = 256
EP = 327680
NCHUNKS = EP // CHUNK
CPT = NCHUNKS // NS
SQ = 16
NBUF = 4
ZCH = 128
ZROWS = 632
AGG_ROWS = NS * ZROWS
WB = 624
WB_LAST = N - (NS - 1) * WB


def _sc_agg_body(mode3, hcat, srcoff, dst2, out, *scr):
    src_st = scr[0]
    dst_st = scr[1]
    bufs = scr[2:2 + NBUF]
    agg_sh = scr[2 + NBUF]
    semg = scr[3 + NBUF:3 + 2 * NBUF]
    sems = scr[3 + 2 * NBUF:3 + 3 * NBUF]

    c = lax.axis_index("c")
    s = lax.axis_index("s")

    if mode3:
        cpt = NCHUNKS // 2 // NS
        sq = SQ // 2
        dst_base = c * (NCHUNKS // 2) + s * cpt
        src_base = dst_base
    else:
        cpt = CPT
        sq = SQ
        dst_base = s * cpt
        src_base = s * cpt

    def zbody(i, carry):
        for j in range(COLH // 16):
            bufs[0][i, pl.ds(j * 16, 16)] = jnp.zeros((16,), jnp.float32)
        return carry
    lax.fori_loop(0, ZCH, zbody, 0)

    zbase = s * ZROWS
    for k in range(ZROWS // ZCH):
        pltpu.sync_copy(bufs[0].at[pl.ds(0, ZCH)],
                        agg_sh.at[pl.ds(zbase + k * ZCH, ZCH)])
    rem = ZROWS % ZCH
    if rem:
        pltpu.sync_copy(bufs[0].at[pl.ds(0, rem)],
                        agg_sh.at[pl.ds(zbase + (ZROWS // ZCH) * ZCH, rem)])
    plsc.subcore_barrier()

    def wait_gather(j, g):
        pltpu.make_async_copy(hcat.at[src_st.at[g]], bufs[j], semg[j]).wait()

    def wait_scatter(j):
        pltpu.make_async_copy(bufs[j], agg_sh.at[dst_st.at[0]], sems[j]).wait()

    pltpu.sync_copy(dst2.at[pl.ds(dst_base, cpt)], dst_st.at[pl.ds(0, cpt)])
    for q in range(cpt // sq):
        pltpu.sync_copy(srcoff.at[pl.ds(src_base + q * sq, sq)],
                        src_st.at[pl.ds(0, sq)])
        if not mode3:
            @pl.when(c == 1)
            def _():
                def addn(r, carry):
                    for k in range(CHUNK // 16):
                        src_st[r, pl.ds(k * 16, 16)] = (
                            src_st[r, pl.ds(k * 16, 16)] + N)
                    return carry
                lax.fori_loop(0, sq, addn, 0)

        def qloop(gg, carry, first=(q == 0)):
            base = gg * NBUF
            for j in range(NBUF):
                if first:
                    @pl.when(gg > 0)
                    def _(j=j):
                        wait_scatter(j)
                else:
                    wait_scatter(j)
                pltpu.async_copy(hcat.at[src_st.at[base + j]], bufs[j], semg[j])
            for j in range(NBUF):
                wait_gather(j, base + j)
                pltpu.async_copy(
                    bufs[j], agg_sh.at[dst_st.at[q * sq + base + j]],
                    sems[j], add=True)
            return carry

        lax.fori_loop(0, sq // NBUF, qloop, 0)

    for j in range(NBUF):
        wait_scatter(j)

    plsc.subcore_barrier()

    @pl.when(s < NS - 1)
    def _():
        pltpu.sync_copy(agg_sh.at[pl.ds(s * WB, WB)],
                        out.at[pl.ds(c * N + s * WB, WB)])

    @pl.when(s == NS - 1)
    def _():
        pltpu.sync_copy(agg_sh.at[pl.ds((NS - 1) * WB, WB_LAST)],
                        out.at[pl.ds(c * N + (NS - 1) * WB, WB_LAST)])


def _sc_agg(hcat, srcoff, dst2, mode3=False):
    mesh = plsc.VectorSubcoreMesh(core_axis_name="c", subcore_axis_name="s",
                                  num_cores=NC, num_subcores=NS)
    return pl.kernel(
        functools.partial(_sc_agg_body, mode3),
        out_type=jax.ShapeDtypeStruct((2 * N, COLH), jnp.float32),
        mesh=mesh,
        compiler_params=pltpu.CompilerParams(use_tc_tiling_on_sc=False),
        scratch_types=[
            pltpu.VMEM((SQ, CHUNK), jnp.int32),
            pltpu.VMEM((CPT, CHUNK), jnp.int32),
        ] + [pltpu.VMEM((CHUNK, COLH), jnp.float32) for _ in range(NBUF)]
        + [pltpu.VMEM_SHARED((AGG_ROWS, COLH), jnp.float32)]
        + [pltpu.SemaphoreType.DMA for _ in range(2 * NBUF)],
    )(hcat, srcoff, dst2)


def _tc_mlp_body(relu, hl_ref, hr_ref, al_ref, ar_ref, w_ref, b_ref, o_ref):
    t = jnp.concatenate(
        [hl_ref[...] + al_ref[...], hr_ref[...] + ar_ref[...]], axis=1)
    y = jnp.dot(t, w_ref[0], preferred_element_type=jnp.float32) + b_ref[0]
    if relu:
        y = jnp.maximum(y, 0.0)
    o_ref[...] = y


def _tc_mlp(hcat, agg, w, b, relu):
    blk = 2000
    nb = N // blk
    return pl.pallas_call(
        functools.partial(_tc_mlp_body, relu),
        grid=(2 * nb,),
        in_specs=[
            pl.BlockSpec((blk, COLH), lambda i: (i % nb, 0)),
            pl.BlockSpec((blk, COLH), lambda i: (nb + i % nb, 0)),
            pl.BlockSpec((blk, COLH), lambda i: (i % nb, 0)),
            pl.BlockSpec((blk, COLH), lambda i: (nb + i % nb, 0)),
            pl.BlockSpec((1, D, COLH), lambda i: (i // nb, 0, 0)),
            pl.BlockSpec((1, 1, COLH), lambda i: (i // nb, 0, 0)),
        ],
        out_specs=pl.BlockSpec((blk, COLH), lambda i: (i, 0)),
        out_shape=jax.ShapeDtypeStruct((2 * N, COLH), jnp.float32),
    )(hcat, hcat, agg, agg,
      jnp.stack([w[:, :COLH], w[:, COLH:]]),
      b.reshape(2, 1, COLH))


def _tc_transform_body(hl_ref, hr_ref, w_ref, o_ref):
    t = jnp.concatenate([hl_ref[...], hr_ref[...]], axis=1)
    o_ref[...] = jnp.dot(t, w_ref[...], preferred_element_type=jnp.float32)


def _tc_transform(hcat, w):
    blk = 2000
    nb = N // blk
    return pl.pallas_call(
        _tc_transform_body,
        grid=(nb,),
        in_specs=[
            pl.BlockSpec((blk, COLH), lambda i: (i, 0)),
            pl.BlockSpec((blk, COLH), lambda i: (nb + i, 0)),
            pl.BlockSpec((D, COLH), lambda i: (0, 0)),
        ],
        out_specs=pl.BlockSpec((blk, COLH), lambda i: (i, 0)),
        out_shape=jax.ShapeDtypeStruct((N, COLH), jnp.float32),
    )(hcat, hcat, w)


def _tc_combine_body(y_ref, a0_ref, a1_ref, b_ref, o_ref):
    o_ref[...] = y_ref[...] + a0_ref[...] + a1_ref[...] + b_ref[...]


def _tc_combine(y, agg, b):
    blk = 2000
    nb = N // blk
    return pl.pallas_call(
        _tc_combine_body,
        grid=(nb,),
        in_specs=[
            pl.BlockSpec((blk, COLH), lambda i: (i, 0)),
            pl.BlockSpec((blk, COLH), lambda i: (i, 0)),
            pl.BlockSpec((blk, COLH), lambda i: (nb + i, 0)),
            pl.BlockSpec((1, COLH), lambda i: (0, 0)),
        ],
        out_specs=pl.BlockSpec((blk, COLH), lambda i: (i, 0)),
        out_shape=jax.ShapeDtypeStruct((N, COLH), jnp.float32),
    )(y, agg, agg, b.reshape(1, COLH))


def kernel(x, edge_index, W1, b1, W2, b2, W3, b3):
    src = edge_index[0].astype(jnp.int32)
    dst = edge_index[1].astype(jnp.int32)
    pad = EP - E
    src2 = jnp.concatenate([src, (jnp.arange(pad, dtype=jnp.int32) * 13) % N]
                           ).reshape(NCHUNKS, CHUNK)
    dst2 = jnp.concatenate([dst, jnp.full((pad,), N, jnp.int32)]).reshape(NCHUNKS, CHUNK)

    hcat = jnp.concatenate([x[:, :COLH], x[:, COLH:]], axis=0)
    agg = _sc_agg(hcat, src2, dst2)
    hcat = _tc_mlp(hcat, agg, W1, b1, relu=True)
    agg = _sc_agg(hcat, src2, dst2)
    hcat = _tc_mlp(hcat, agg, W2, b2, relu=True)
    w3p = jnp.zeros((D, COLH), jnp.float32).at[:, :40].set(W3)
    b3p = jnp.zeros((COLH,), jnp.float32).at[:40].set(b3)
    y = _tc_transform(hcat, w3p)
    agg = _sc_agg(y, src2, dst2, mode3=True)
    out = _tc_combine(y, agg, b3p)
    return out[:, :40]
